# Initial kernel scaffold; baseline (speedup 1.0000x reference)
import jax
import jax.numpy as jnp
from jax.experimental import pallas as pl


def kernel(edge_index, u_emb, i_emb, W_gc, b_gc, W_bi, b_bi):
    raise NotImplementedError("write your pallas kernel here")


# speedup vs baseline: 17.0257x; 17.0257x over previous
import functools

import jax
import jax.numpy as jnp
from jax import lax
from jax.experimental import pallas as pl
from jax.experimental.pallas import tpu as pltpu
from jax.experimental.pallas import tpu_sc as plsc

_NS = 16
_CHUNK = 128
_BLK = 512


def _ceil_to(x, m):
    return (x + m - 1) // m * m


def _make_spmm(n2, nup, d, nch):
    rows_pt = nup // _NS
    mesh = plsc.VectorSubcoreMesh(core_axis_name="c", subcore_axis_name="s")

    @functools.partial(
        pl.kernel,
        mesh=mesh,
        out_type=jax.ShapeDtypeStruct((n2, d), jnp.float32),
        scratch_types=[
            pltpu.VMEM_SHARED((nup, d), jnp.float32),
            pltpu.VMEM((2, _CHUNK), jnp.int32),
            pltpu.VMEM((_CHUNK, d), jnp.float32),
            pltpu.SemaphoreType.DMA,
        ],
        compiler_params=pltpu.CompilerParams(use_tc_tiling_on_sc=False),
    )
    def spmm(xp, cidx, out, acc, cbuf, rbuf, sem):
        c = lax.axis_index("c")
        s = lax.axis_index("s")
        r0 = s * rows_pt
        base = c * nup + r0
        pltpu.sync_copy(xp.at[pl.ds(base, rows_pt)], acc.at[pl.ds(r0, rows_pt)])
        plsc.subcore_barrier()

        def body(k, carry):
            pltpu.sync_copy(cidx.at[c, s, k], cbuf)
            pltpu.async_copy(xp.at[cbuf.at[0]], rbuf, sem).wait()
            pltpu.sync_copy(rbuf, acc.at[cbuf.at[1]], add=True)
            return carry

        lax.fori_loop(0, nch, body, 0)
        plsc.subcore_barrier()
        pltpu.sync_copy(acc.at[pl.ds(r0, rows_pt)], out.at[pl.ds(base, rows_pt)])

    return spmm


def _pre_body(deg_ref, x_ref, dinv_ref, xp_ref):
    dinv = lax.rsqrt(jnp.maximum(deg_ref[...], 1.0))
    dinv_ref[...] = dinv
    xp_ref[...] = x_ref[...] * dinv


def _dense_body(acc_ref, x_ref, dinv_ref, mean_ref, wg_ref, bg_ref,
                wb_ref, bb_ref, xn_ref, xpn_ref, mout_ref):
    dinv = dinv_ref[...]
    g = acc_ref[...] * dinv
    x = x_ref[...]
    h1 = jnp.dot(g, wg_ref[...], preferred_element_type=jnp.float32,
                 precision=lax.Precision.HIGHEST) + bg_ref[...]
    s_e = jnp.where(h1 >= 0, h1, 0.2 * h1)
    h2 = jnp.dot(x * g, wb_ref[...], preferred_element_type=jnp.float32,
                 precision=lax.Precision.HIGHEST) + bb_ref[...]
    b_e = jnp.where(h2 >= 0, h2, 0.2 * h2)
    xn = s_e + b_e
    nrm = jnp.sqrt(jnp.sum(xn * xn, axis=1, keepdims=True))
    xn = xn / jnp.maximum(nrm, 1e-12)
    xn_ref[...] = xn
    xpn_ref[...] = xn * dinv
    mout_ref[...] = mean_ref[...] + xn


def _row_spec(d):
    return pl.BlockSpec((_BLK, d), lambda i: (i, 0))


def _full_spec(shape):
    return pl.BlockSpec(shape, lambda i: (0,) * len(shape))


def _pre_call(deg, x0, n2, d):
    grid = (n2 // _BLK,)
    return pl.pallas_call(
        _pre_body,
        grid=grid,
        in_specs=[_row_spec(1), _row_spec(d)],
        out_specs=[_row_spec(1), _row_spec(d)],
        out_shape=[jax.ShapeDtypeStruct((n2, 1), jnp.float32),
                   jax.ShapeDtypeStruct((n2, d), jnp.float32)],
    )(deg, x0)


def _dense_call(accv, x, dinv, mean, wgt, bg, wbt, bb, n2, d):
    grid = (n2 // _BLK,)
    return pl.pallas_call(
        _dense_body,
        grid=grid,
        in_specs=[_row_spec(d), _row_spec(d), _row_spec(1), _row_spec(d),
                  _full_spec((d, d)), _full_spec((1, d)),
                  _full_spec((d, d)), _full_spec((1, d))],
        out_specs=[_row_spec(d), _row_spec(d), _row_spec(d)],
        out_shape=[jax.ShapeDtypeStruct((n2, d), jnp.float32),
                   jax.ShapeDtypeStruct((n2, d), jnp.float32),
                   jax.ShapeDtypeStruct((n2, d), jnp.float32)],
    )(accv, x, dinv, mean, wgt, bg, wbt, bb)


def kernel(edge_index, u_emb, i_emb, W_gc, b_gc, W_bi, b_bi):
    nu = u_emb.shape[0]
    ni = i_emb.shape[0]
    d = u_emb.shape[1]
    e = edge_index.shape[1]
    layers = W_gc.shape[0]

    nup = _ceil_to(max(nu, ni), _BLK)
    n2 = 2 * nup
    ept = _ceil_to(-(-e // _NS), _CHUNK)
    nch = ept // _CHUNK

    src = edge_index[0].astype(jnp.int32)
    dst = edge_index[1].astype(jnp.int32)

    def _laid(idx, padval):
        p = jnp.full((_NS * ept,), padval, jnp.int32).at[:e].set(idx)
        return p.reshape(_NS, nch, _CHUNK)

    cidx = jnp.stack([
        jnp.stack([_laid(nup + dst, 0), _laid(src, nup - 1)], axis=2),
        jnp.stack([_laid(src, 0), _laid(dst, nup - 1)], axis=2),
    ])

    x0 = jnp.zeros((n2, d), jnp.float32)
    x0 = lax.dynamic_update_slice(x0, u_emb.astype(jnp.float32), (0, 0))
    x0 = lax.dynamic_update_slice(x0, i_emb.astype(jnp.float32), (nup, 0))

    spmm = _make_spmm(n2, nup, d, nch)

    deg = spmm(jnp.ones((n2, d), jnp.float32), cidx)[:, :1]
    dinv, xp = _pre_call(deg, x0, n2, d)

    x = x0
    mean = x0
    for l in range(layers):
        accv = spmm(xp, cidx)
        x, xp, mean = _dense_call(
            accv, x, dinv, mean,
            W_gc[l].T, b_gc[l][None, :], W_bi[l].T, b_bi[l][None, :],
            n2, d)

    embs = mean * (1.0 / (layers + 1))
    return embs[:nu], embs[nup:nup + ni]


# speedup vs baseline: 26.1537x; 1.5361x over previous
import functools

import jax
import jax.numpy as jnp
from jax import lax
from jax.experimental import pallas as pl
from jax.experimental.pallas import tpu as pltpu
from jax.experimental.pallas import tpu_sc as plsc

_NS = 16
_CHUNK = 128
_BLK = 512


def _ceil_to(x, m):
    return (x + m - 1) // m * m


def _make_spmm(n2, nup, d, nch):
    rows_pt = nup // _NS
    mesh = plsc.VectorSubcoreMesh(core_axis_name="c", subcore_axis_name="s")

    @functools.partial(
        pl.kernel,
        mesh=mesh,
        out_type=jax.ShapeDtypeStruct((n2, d), jnp.float32),
        scratch_types=[
            pltpu.VMEM_SHARED((nup, d), jnp.float32),
            pltpu.VMEM((2, _CHUNK), jnp.int32),
            pltpu.VMEM((2, _CHUNK), jnp.int32),
            pltpu.VMEM((_CHUNK, d), jnp.float32),
            pltpu.VMEM((_CHUNK, d), jnp.float32),
            pltpu.SemaphoreType.DMA,
            pltpu.SemaphoreType.DMA,
            pltpu.SemaphoreType.DMA,
            pltpu.SemaphoreType.DMA,
            pltpu.SemaphoreType.DMA,
            pltpu.SemaphoreType.DMA,
        ],
        compiler_params=pltpu.CompilerParams(use_tc_tiling_on_sc=False),
    )
    def spmm(xp, cidx, out, acc, cbuf0, cbuf1, rbuf0, rbuf1,
             si0, si1, sg0, sg1, ss0, ss1):
        c = lax.axis_index("c")
        s = lax.axis_index("s")
        r0 = s * rows_pt
        base = c * nup + r0
        cbuf = (cbuf0, cbuf1)
        rbuf = (rbuf0, rbuf1)
        si = (si0, si1)
        sg = (sg0, sg1)
        ss = (ss0, ss1)

        def start_i(k, b):
            pltpu.async_copy(cidx.at[c, s, k], cbuf[b], si[b])

        def wait_i(k, b):
            pltpu.make_async_copy(cidx.at[c, s, k], cbuf[b], si[b]).wait()

        def start_g(b):
            pltpu.async_copy(xp.at[cbuf[b].at[0]], rbuf[b], sg[b])

        def wait_g(b):
            pltpu.make_async_copy(xp.at[cbuf[b].at[0]], rbuf[b], sg[b]).wait()

        def start_s(b):
            pltpu.async_copy(rbuf[b], acc.at[cbuf[b].at[1]], ss[b], add=True)

        def wait_s(b):
            pltpu.make_async_copy(rbuf[b], acc.at[cbuf[b].at[1]], ss[b]).wait()

        start_i(0, 0)
        pltpu.sync_copy(xp.at[pl.ds(base, rows_pt)], acc.at[pl.ds(r0, rows_pt)])
        plsc.subcore_barrier()

        def step(k, b, first):
            bn = 1 - b
            wait_i(k, b)
            start_g(b)
            if not first:
                wait_s(bn)

            @pl.when(k + 1 < nch)
            def _():
                start_i(k + 1, bn)

            wait_g(b)
            start_s(b)

        step(0, 0, True)

        def pair(t, carry):
            k = 1 + 2 * t
            step(k, 1, False)
            step(k + 1, 0, False)
            return carry

        npairs = (nch - 1) // 2
        lax.fori_loop(0, npairs, pair, 0)
        tail = 1 + 2 * npairs
        if tail < nch:
            step(tail, tail % 2, False)
        wait_s((nch - 1) % 2)

        plsc.subcore_barrier()
        pltpu.sync_copy(acc.at[pl.ds(r0, rows_pt)], out.at[pl.ds(base, rows_pt)])

    return spmm


def _pre_body(deg_ref, x_ref, dinv_ref, xp_ref):
    dinv = lax.rsqrt(jnp.maximum(deg_ref[...], 1.0))
    dinv_ref[...] = dinv
    xp_ref[...] = x_ref[...] * dinv


def _dense_body(acc_ref, x_ref, dinv_ref, mean_ref, wg_ref, bg_ref,
                wb_ref, bb_ref, xn_ref, xpn_ref, mout_ref):
    dinv = dinv_ref[...]
    g = acc_ref[...] * dinv
    x = x_ref[...]
    h1 = jnp.dot(g, wg_ref[...], preferred_element_type=jnp.float32,
                 precision=lax.Precision.HIGHEST) + bg_ref[...]
    s_e = jnp.where(h1 >= 0, h1, 0.2 * h1)
    h2 = jnp.dot(x * g, wb_ref[...], preferred_element_type=jnp.float32,
                 precision=lax.Precision.HIGHEST) + bb_ref[...]
    b_e = jnp.where(h2 >= 0, h2, 0.2 * h2)
    xn = s_e + b_e
    nrm = jnp.sqrt(jnp.sum(xn * xn, axis=1, keepdims=True))
    xn = xn / jnp.maximum(nrm, 1e-12)
    xn_ref[...] = xn
    xpn_ref[...] = xn * dinv
    mout_ref[...] = mean_ref[...] + xn


def _row_spec(d):
    return pl.BlockSpec((_BLK, d), lambda i: (i, 0))


def _full_spec(shape):
    return pl.BlockSpec(shape, lambda i: (0,) * len(shape))


def _pre_call(deg, x0, n2, d):
    grid = (n2 // _BLK,)
    return pl.pallas_call(
        _pre_body,
        grid=grid,
        in_specs=[_row_spec(1), _row_spec(d)],
        out_specs=[_row_spec(1), _row_spec(d)],
        out_shape=[jax.ShapeDtypeStruct((n2, 1), jnp.float32),
                   jax.ShapeDtypeStruct((n2, d), jnp.float32)],
    )(deg, x0)


def _dense_call(accv, x, dinv, mean, wgt, bg, wbt, bb, n2, d):
    grid = (n2 // _BLK,)
    return pl.pallas_call(
        _dense_body,
        grid=grid,
        in_specs=[_row_spec(d), _row_spec(d), _row_spec(1), _row_spec(d),
                  _full_spec((d, d)), _full_spec((1, d)),
                  _full_spec((d, d)), _full_spec((1, d))],
        out_specs=[_row_spec(d), _row_spec(d), _row_spec(d)],
        out_shape=[jax.ShapeDtypeStruct((n2, d), jnp.float32),
                   jax.ShapeDtypeStruct((n2, d), jnp.float32),
                   jax.ShapeDtypeStruct((n2, d), jnp.float32)],
    )(accv, x, dinv, mean, wgt, bg, wbt, bb)


def kernel(edge_index, u_emb, i_emb, W_gc, b_gc, W_bi, b_bi):
    nu = u_emb.shape[0]
    ni = i_emb.shape[0]
    d = u_emb.shape[1]
    e = edge_index.shape[1]
    layers = W_gc.shape[0]

    nup = _ceil_to(max(nu, ni), _BLK)
    n2 = 2 * nup
    ept = _ceil_to(-(-e // _NS), _CHUNK)
    nch = ept // _CHUNK

    src = edge_index[0].astype(jnp.int32)
    dst = edge_index[1].astype(jnp.int32)

    def _laid(idx, padval):
        p = jnp.full((_NS * ept,), padval, jnp.int32).at[:e].set(idx)
        return p.reshape(_NS, nch, _CHUNK)

    cidx = jnp.stack([
        jnp.stack([_laid(nup + dst, 0), _laid(src, nup - 1)], axis=2),
        jnp.stack([_laid(src, 0), _laid(dst, nup - 1)], axis=2),
    ])

    x0 = jnp.zeros((n2, d), jnp.float32)
    x0 = lax.dynamic_update_slice(x0, u_emb.astype(jnp.float32), (0, 0))
    x0 = lax.dynamic_update_slice(x0, i_emb.astype(jnp.float32), (nup, 0))

    spmm = _make_spmm(n2, nup, d, nch)

    deg = spmm(jnp.ones((n2, d), jnp.float32), cidx)[:, :1]
    dinv, xp = _pre_call(deg, x0, n2, d)

    x = x0
    mean = x0
    for l in range(layers):
        accv = spmm(xp, cidx)
        x, xp, mean = _dense_call(
            accv, x, dinv, mean,
            W_gc[l].T, b_gc[l][None, :], W_bi[l].T, b_bi[l][None, :],
            n2, d)

    embs = mean * (1.0 / (layers + 1))
    return embs[:nu], embs[nup:nup + ni]
